# baseline (device time: 67348 ns/iter reference)
import jax
import jax.numpy as jnp
from jax import lax
from jax.experimental import pallas as pl
from jax.experimental.pallas import tpu as pltpu

N_DEV = 4
SQ = 1024
SKV = 1024
H_LOC = 8
DH = 128
D_MODEL = 1024
D_QKV = H_LOC * DH
SCALE = 0.08838834764831843
BLK = 64
CHUNK = SQ // N_DEV


def kernel(x, Wq, K_ext, V_ext, Wo):
    my_pos = lax.axis_index("i")

    x2 = x[0].astype(jnp.bfloat16)
    Wq_loc = (lax.dynamic_slice(
        Wq, (0, my_pos * D_QKV), (D_MODEL, D_QKV)) * SCALE).astype(jnp.bfloat16)
    Wo_loc = lax.dynamic_slice(
        Wo, (my_pos * D_QKV, 0), (D_QKV, D_MODEL)).astype(jnp.bfloat16)
    K = jnp.transpose(K_ext[0], (1, 0, 2)).astype(jnp.bfloat16)
    V = jnp.transpose(V_ext[0], (1, 0, 2)).astype(jnp.bfloat16)

    def body(x_ref, wq_ref, k_ref, v_ref, wo_ref, out_ref,
             stage_ref, rs_ref, ag_ref,
             rs_send_sems, rs_recv_sems, ag_send_sems, ag_recv_sems):
        my = lax.axis_index("i")

        barrier_sem = pltpu.get_barrier_semaphore()
        for d in range(1, N_DEV):
            pl.semaphore_signal(
                barrier_sem, inc=1,
                device_id=(lax.rem(my + d, N_DEV),),
                device_id_type=pl.DeviceIdType.MESH,
            )
        pl.semaphore_wait(barrier_sem, N_DEV - 1)

        def rs_send_desc(c):
            return pltpu.make_async_remote_copy(
                src_ref=stage_ref.at[c],
                dst_ref=rs_ref.at[my],
                send_sem=rs_send_sems.at[c],
                recv_sem=rs_recv_sems.at[my],
                device_id=(c,),
                device_id_type=pl.DeviceIdType.MESH,
            )

        def rs_recv_desc(s):
            return pltpu.make_async_remote_copy(
                src_ref=stage_ref.at[s],
                dst_ref=rs_ref.at[s],
                send_sem=rs_send_sems.at[s],
                recv_sem=rs_recv_sems.at[s],
                device_id=(s,),
                device_id_type=pl.DeviceIdType.MESH,
            )

        def ag_send_desc(c, d):
            return pltpu.make_async_remote_copy(
                src_ref=ag_ref.at[c],
                dst_ref=ag_ref.at[c],
                send_sem=ag_send_sems.at[d],
                recv_sem=ag_recv_sems.at[c],
                device_id=(d,),
                device_id_type=pl.DeviceIdType.MESH,
            )

        def reduce_and_broadcast(cc):
            for s_id in range(N_DEV):
                if s_id != cc:
                    rs_recv_desc(s_id).wait_recv()
            red = (rs_ref[0].astype(jnp.float32)
                   + rs_ref[1].astype(jnp.float32)
                   + rs_ref[2].astype(jnp.float32)
                   + rs_ref[3].astype(jnp.float32))
            ag_ref[cc] = red.astype(jnp.bfloat16)
            for d in range(N_DEV):
                if d != cc:
                    ag_send_desc(cc, d).start()

        rb = lax.broadcasted_iota(jnp.int32, (CHUNK, CHUNK), 0) // BLK
        cb = lax.broadcasted_iota(jnp.int32, (CHUNK, CHUNK), 1) // BLK
        diag_bias = jnp.where(cb <= rb, 0.0, -1e9).astype(jnp.float32)

        for c in range(N_DEV):
            kvlen = (c + 1) * CHUNK
            xc = x_ref[pl.ds(c * CHUNK, CHUNK), :]
            qc = jax.lax.dot(xc, wq_ref[...],
                             preferred_element_type=jnp.float32
                             ).astype(jnp.bfloat16)

            ctx_cols = []
            for h in range(H_LOC):
                qh = qc[:, h * DH:(h + 1) * DH]
                vh = v_ref[h, pl.ds(0, kvlen), :]
                kd = k_ref[h, pl.ds(c * CHUNK, CHUNK), :]
                sd = lax.dot_general(
                    qh, kd, (((1,), (1,)), ((), ())),
                    preferred_element_type=jnp.float32)
                wd = jnp.exp(sd + diag_bias)
                if c > 0:
                    kf = k_ref[h, pl.ds(0, c * CHUNK), :]
                    sf = lax.dot_general(
                        qh, kf, (((1,), (1,)), ((), ())),
                        preferred_element_type=jnp.float32)
                    w = jnp.concatenate([jnp.exp(sf), wd], axis=1)
                else:
                    w = wd
                denom = jnp.sum(w, axis=-1, keepdims=True)
                ctx_raw = jax.lax.dot(
                    w.astype(jnp.bfloat16), vh,
                    preferred_element_type=jnp.float32)
                ctx_cols.append(ctx_raw * (1.0 / denom))
            ctx = jnp.concatenate(ctx_cols, axis=1).astype(jnp.bfloat16)
            pc = jax.lax.dot(ctx, wo_ref[...],
                             preferred_element_type=jnp.float32)
            pcb = pc.astype(jnp.bfloat16)

            @pl.when(c == my)
            def _():
                rs_ref[c] = pcb

            @pl.when(c != my)
            def _():
                stage_ref[c] = pcb
                rs_send_desc(c).start()

            if c >= 1:
                @pl.when(c - 1 == my)
                def _():
                    reduce_and_broadcast(c - 1)

        @pl.when(my == N_DEV - 1)
        def _():
            reduce_and_broadcast(N_DEV - 1)

        for j in range(N_DEV):
            @pl.when(j != my)
            def _():
                pltpu.make_async_remote_copy(
                    src_ref=ag_ref.at[j],
                    dst_ref=ag_ref.at[j],
                    send_sem=ag_send_sems.at[j],
                    recv_sem=ag_recv_sems.at[j],
                    device_id=(j,),
                    device_id_type=pl.DeviceIdType.MESH,
                ).wait_recv()
                out_ref[0, pl.ds(j * CHUNK, CHUNK), :] = (
                    ag_ref[j].astype(jnp.float32))

            @pl.when(j == my)
            def _():
                out_ref[0, pl.ds(j * CHUNK, CHUNK), :] = (
                    ag_ref[j].astype(jnp.float32))

        for c in range(N_DEV):
            @pl.when(c != my)
            def _():
                rs_send_desc(c).wait_send()
        for d in range(N_DEV):
            @pl.when(d != my)
            def _():
                ag_send_desc(my, d).wait_send()

    return pl.pallas_call(
        body,
        out_shape=jax.ShapeDtypeStruct((1, SQ, D_MODEL), jnp.float32),
        in_specs=[pl.BlockSpec(memory_space=pltpu.VMEM)] * 5,
        out_specs=pl.BlockSpec(memory_space=pltpu.VMEM),
        scratch_shapes=[
            pltpu.VMEM((N_DEV, CHUNK, D_MODEL), jnp.bfloat16),
            pltpu.VMEM((N_DEV, CHUNK, D_MODEL), jnp.bfloat16),
            pltpu.VMEM((N_DEV, CHUNK, D_MODEL), jnp.bfloat16),
            pltpu.SemaphoreType.DMA((N_DEV,)),
            pltpu.SemaphoreType.DMA((N_DEV,)),
            pltpu.SemaphoreType.DMA((N_DEV,)),
            pltpu.SemaphoreType.DMA((N_DEV,)),
        ],
        compiler_params=pltpu.CompilerParams(collective_id=0),
    )(x2, Wq_loc, K, V, Wo_loc)
